# baseline (speedup 1.0000x reference)
import functools

import jax
import jax.numpy as jnp
from jax import lax
from jax.experimental import pallas as pl
from jax.experimental.pallas import tpu as pltpu
from jax.experimental.pallas import tpu_sc as plsc

NC = 2
NS = 16
NW = NC * NS
CH = 128
G = 80
L = 16


def _sc_aggregate(x, src_p, dst_p, z_agg):
    n, d = x.shape
    rpt = (n // NS) // 8 * 8
    rlast = n - (NS - 1) * rpt

    mesh = plsc.VectorSubcoreMesh(core_axis_name="c", subcore_axis_name="s")

    @functools.partial(
        pl.kernel,
        mesh=mesh,
        out_type=[
            jax.ShapeDtypeStruct((NC, n, d), jnp.float32),
            jax.ShapeDtypeStruct((NC, n, d), jnp.float32),
        ],
        scratch_types=[
            pltpu.VMEM((G // 2, CH), jnp.int32),
            pltpu.VMEM((G, CH), jnp.int32),
            pltpu.VMEM((CH, d), jnp.float32),
            pltpu.VMEM((CH, d), jnp.float32),
            pltpu.VMEM_SHARED((n + 8, d), jnp.float32),
            pltpu.SemaphoreType.DMA,
            pltpu.SemaphoreType.DMA,
            pltpu.SemaphoreType.DMA,
            pltpu.SemaphoreType.DMA,
        ],
    )
    def sc_kern(x_hbm, src_hbm, dst_hbm, zagg_hbm,
                agg_out, deg_out, idx_s, idx_d, rows0, rows1,
                agg_sh, sem0, sem1, semA, semB):
        c = lax.axis_index("c")
        s = lax.axis_index("s")
        r0 = s * rpt
        wid = c * NS + s

        def zero_acc():
            @pl.when(s < NS - 1)
            def _():
                pltpu.sync_copy(zagg_hbm.at[pl.ds(r0, rpt)],
                                agg_sh.at[pl.ds(r0, rpt)])

            @pl.when(s == NS - 1)
            def _():
                pltpu.sync_copy(zagg_hbm.at[pl.ds(r0, rlast)],
                                agg_sh.at[pl.ds(r0, rlast)])

        def writeback(out_ref):
            @pl.when(s < NS - 1)
            def _():
                pltpu.sync_copy(agg_sh.at[pl.ds(r0, rpt)],
                                out_ref.at[c, pl.ds(r0, rpt)])

            @pl.when(s == NS - 1)
            def _():
                pltpu.sync_copy(agg_sh.at[pl.ds(r0, rlast)],
                                out_ref.at[c, pl.ds(r0, rlast)])

        zero_acc()
        pltpu.sync_copy(dst_hbm.at[pl.ds(wid * G, G)], idx_d)
        plsc.subcore_barrier()

        H = G // 2

        def gather(g, buf, sem):
            return pltpu.async_copy(x_hbm.at[idx_s.at[g]], buf, sem)

        def gather_wait(buf, sem):
            pltpu.make_async_copy(x_hbm.at[idx_s.at[0]], buf, sem).wait()

        for h in range(2):
            pltpu.sync_copy(src_hbm.at[pl.ds(wid * G + h * H, H)], idx_s)
            gather(0, rows0, sem0)

            def body1(gg, carry):
                g0 = 2 * gg
                gd = h * H + g0
                gather_wait(rows0, sem0)
                gather(g0 + 1, rows1, sem1)
                pltpu.sync_copy(rows0, agg_sh.at[idx_d.at[gd]], add=True)
                gather_wait(rows1, sem1)
                gather(lax.rem(g0 + 2, H), rows0, sem0)
                pltpu.sync_copy(rows1, agg_sh.at[idx_d.at[gd + 1]], add=True)
                return carry

            lax.fori_loop(0, H // 2, body1, 0)
            gather_wait(rows0, sem0)

        plsc.subcore_barrier()

        writeback(agg_out)
        zero_acc()

        def fill_row(i, carry):
            def fill_lane(j, carry2):
                rows0[i, pl.ds(j * L, L)] = jnp.full((L,), 1.0, jnp.float32)
                return carry2
            return lax.fori_loop(0, d // L, fill_lane, carry)

        lax.fori_loop(0, CH, fill_row, 0)
        plsc.subcore_barrier()

        def body2(gg, carry):
            g0 = 2 * gg
            pltpu.async_copy(rows0, agg_sh.at[idx_d.at[g0]], semA, add=True)
            pltpu.async_copy(rows0, agg_sh.at[idx_d.at[g0 + 1]], semB,
                             add=True)
            pltpu.make_async_copy(rows0, agg_sh.at[idx_d.at[g0]],
                                  semA).wait()
            pltpu.make_async_copy(rows0, agg_sh.at[idx_d.at[g0 + 1]],
                                  semB).wait()
            return carry

        lax.fori_loop(0, G // 2, body2, 0)
        plsc.subcore_barrier()

        writeback(deg_out)

    return sc_kern(x, src_p, dst_p, z_agg)


def _tc_linear_stats(agg_p, deg_p, W, block):
    _, n, d = agg_p.shape
    nb = n // block

    def body(p_ref, d_ref, w_ref, h_ref, s_ref, q_ref):
        i = pl.program_id(0)
        agg = p_ref[0] + p_ref[1]
        deg = d_ref[0, :, 0:1] + d_ref[1, :, 0:1]
        deg = jnp.maximum(deg, 1.0)
        h = jnp.dot(agg / deg, w_ref[...], preferred_element_type=jnp.float32)
        h_ref[...] = h
        s8 = jnp.sum(h.reshape(block // 8, 8, d), axis=0)
        q8 = jnp.sum((h * h).reshape(block // 8, 8, d), axis=0)

        @pl.when(i == 0)
        def _():
            s_ref[...] = s8
            q_ref[...] = q8

        @pl.when(i != 0)
        def _():
            s_ref[...] += s8
            q_ref[...] += q8

    return pl.pallas_call(
        body,
        grid=(nb,),
        in_specs=[
            pl.BlockSpec((NC, block, d), lambda i: (0, i, 0)),
            pl.BlockSpec((NC, block, d), lambda i: (0, i, 0)),
            pl.BlockSpec((d, d), lambda i: (0, 0)),
        ],
        out_specs=[
            pl.BlockSpec((block, d), lambda i: (i, 0)),
            pl.BlockSpec((8, d), lambda i: (0, 0)),
            pl.BlockSpec((8, d), lambda i: (0, 0)),
        ],
        out_shape=[
            jax.ShapeDtypeStruct((n, d), jnp.float32),
            jax.ShapeDtypeStruct((8, d), jnp.float32),
            jax.ShapeDtypeStruct((8, d), jnp.float32),
        ],
    )(agg_p, deg_p, W)


def _tc_bn_gru(h_pre, h_prev, sums, sumsq, gamma, beta,
               Wz, Uz, bz, Wr, Ur, br, Wh, Uh, bh, block):
    n, d = h_pre.shape
    nb = n // block
    inv_n = 1.0 / n

    def body(h_ref, hp_ref, s_ref, q_ref, g_ref, b_ref,
             wz_ref, uz_ref, bz_ref, wr_ref, ur_ref, br_ref,
             wh_ref, uh_ref, bh_ref, o_ref):
        mu = jnp.sum(s_ref[...], axis=0, keepdims=True) * inv_n
        ex2 = jnp.sum(q_ref[...], axis=0, keepdims=True) * inv_n
        var = ex2 - mu * mu
        inv = lax.rsqrt(var + 1e-5)
        h = (h_ref[...] - mu) * (inv * g_ref[...]) + b_ref[...]
        h = jnp.maximum(h, 0.0)
        hp = hp_ref[...]
        dot = lambda a, b: jnp.dot(a, b, preferred_element_type=jnp.float32)
        z = jax.nn.sigmoid(dot(h, wz_ref[...]) + dot(hp, uz_ref[...])
                           + bz_ref[...])
        r = jax.nn.sigmoid(dot(h, wr_ref[...]) + dot(hp, ur_ref[...])
                           + br_ref[...])
        ht = jnp.tanh(dot(h, wh_ref[...]) + dot(r * hp, uh_ref[...])
                      + bh_ref[...])
        o_ref[...] = (1.0 - z) * hp + z * ht

    blk = pl.BlockSpec((block, d), lambda i: (i, 0))
    small = pl.BlockSpec((8, d), lambda i: (0, 0))
    row = pl.BlockSpec((1, d), lambda i: (0, 0))
    mat = pl.BlockSpec((d, d), lambda i: (0, 0))
    return pl.pallas_call(
        body,
        grid=(nb,),
        in_specs=[blk, blk, small, small, row, row,
                  mat, mat, row, mat, mat, row, mat, mat, row],
        out_specs=blk,
        out_shape=jax.ShapeDtypeStruct((n, d), jnp.float32),
    )(h_pre, h_prev, sums, sumsq, gamma.reshape(1, d), beta.reshape(1, d),
      Wz, Uz, bz.reshape(1, d), Wr, Ur, br.reshape(1, d),
      Wh, Uh, bh.reshape(1, d))


def kernel(x, edge_index, h_prev, W, gamma, beta,
           Wz, Uz, bz, Wr, Ur, br, Wh, Uh, bh):
    n, d = x.shape
    e = edge_index.shape[1]
    cap = NW * G * CH
    assert e <= cap and n % 8 == 0

    src = edge_index[0].astype(jnp.int32)
    dst = edge_index[1].astype(jnp.int32)
    src_p = jnp.concatenate(
        [src, jnp.zeros((cap - e,), jnp.int32)]).reshape(NW * G, CH)
    dst_p = jnp.concatenate(
        [dst, jnp.full((cap - e,), n, jnp.int32)]).reshape(NW * G, CH)
    z_agg = jnp.zeros((n, d), jnp.float32)

    agg_p, deg_p = _sc_aggregate(x, src_p, dst_p, z_agg)

    block = 1000
    h_pre, sums, sumsq = _tc_linear_stats(agg_p, deg_p, W, block)
    return _tc_bn_gru(h_pre, h_prev, sums, sumsq, gamma, beta,
                      Wz, Uz, bz, Wr, Ur, br, Wh, Uh, bh, block)


# speedup vs baseline: 1.7439x; 1.7439x over previous
import functools

import jax
import jax.numpy as jnp
from jax import lax
from jax.experimental import pallas as pl
from jax.experimental.pallas import tpu as pltpu
from jax.experimental.pallas import tpu_sc as plsc

NC = 2
NS = 16
NW = NC * NS
C = 80
L = 16


def _sc_aggregate(x, src, dst, z_agg):
    n, d = x.shape
    e = src.shape[0]
    ept = e // NW
    gc = ept // C
    rpt = (n // NS) // 8 * 8
    rlast = n - (NS - 1) * rpt
    assert gc % 2 == 1

    mesh = plsc.VectorSubcoreMesh(core_axis_name="c", subcore_axis_name="s")

    @functools.partial(
        pl.kernel,
        mesh=mesh,
        out_type=[
            jax.ShapeDtypeStruct((NC, n, d), jnp.float32),
            jax.ShapeDtypeStruct((NC, n, d), jnp.float32),
        ],
        scratch_types=[
            pltpu.VMEM((C,), jnp.int32),
            pltpu.VMEM((C,), jnp.int32),
            pltpu.VMEM((C,), jnp.int32),
            pltpu.VMEM((C,), jnp.int32),
            pltpu.VMEM((C, d), jnp.float32),
            pltpu.VMEM((C, d), jnp.float32),
            pltpu.VMEM((C, d), jnp.float32),
            pltpu.VMEM_SHARED((n, d), jnp.float32),
            pltpu.SemaphoreType.DMA,
            pltpu.SemaphoreType.DMA,
        ],
    )
    def sc_kern(x_hbm, src_hbm, dst_hbm, zagg_hbm, agg_out, deg_out,
                idx_s0, idx_s1, idx_d0, idx_d1, rows0, rows1, ones_v,
                agg_sh, sem0, sem1):
        c = lax.axis_index("c")
        s = lax.axis_index("s")
        r0 = s * rpt
        base = (c * NS + s) * ept

        def zero_acc():
            @pl.when(s < NS - 1)
            def _():
                pltpu.sync_copy(zagg_hbm.at[pl.ds(r0, rpt)],
                                agg_sh.at[pl.ds(r0, rpt)])

            @pl.when(s == NS - 1)
            def _():
                pltpu.sync_copy(zagg_hbm.at[pl.ds(r0, rlast)],
                                agg_sh.at[pl.ds(r0, rlast)])

        def writeback(out_ref):
            @pl.when(s < NS - 1)
            def _():
                pltpu.sync_copy(agg_sh.at[pl.ds(r0, rpt)],
                                out_ref.at[c, pl.ds(r0, rpt)])

            @pl.when(s == NS - 1)
            def _():
                pltpu.sync_copy(agg_sh.at[pl.ds(r0, rlast)],
                                out_ref.at[c, pl.ds(r0, rlast)])

        zero_acc()

        def fill_row(i, carry):
            def fill_lane(j, carry2):
                ones_v[i, pl.ds(j * L, L)] = jnp.full((L,), 1.0, jnp.float32)
                return carry2
            return lax.fori_loop(0, d // L, fill_lane, carry)

        lax.fori_loop(0, C, fill_row, 0)
        plsc.subcore_barrier()

        pltpu.sync_copy(src_hbm.at[pl.ds(base, C)], idx_s0)
        pltpu.sync_copy(dst_hbm.at[pl.ds(base, C)], idx_d0)
        pltpu.async_copy(x_hbm.at[idx_s0], rows0, sem0)

        def body1(gg, carry):
            off1 = base + (2 * gg + 1) * C
            off2 = base + (2 * gg + 2) * C
            pltpu.sync_copy(src_hbm.at[pl.ds(off1, C)], idx_s1)
            pltpu.sync_copy(dst_hbm.at[pl.ds(off1, C)], idx_d1)
            pltpu.async_copy(x_hbm.at[idx_s1], rows1, sem1)
            pltpu.make_async_copy(x_hbm.at[idx_s0], rows0, sem0).wait()
            pltpu.sync_copy(rows0, agg_sh.at[idx_d0], add=True)
            pltpu.sync_copy(src_hbm.at[pl.ds(off2, C)], idx_s0)
            pltpu.sync_copy(dst_hbm.at[pl.ds(off2, C)], idx_d0)
            pltpu.async_copy(x_hbm.at[idx_s0], rows0, sem0)
            pltpu.make_async_copy(x_hbm.at[idx_s1], rows1, sem1).wait()
            pltpu.sync_copy(rows1, agg_sh.at[idx_d1], add=True)
            return carry

        lax.fori_loop(0, (gc - 1) // 2, body1, 0)
        pltpu.make_async_copy(x_hbm.at[idx_s0], rows0, sem0).wait()
        pltpu.sync_copy(rows0, agg_sh.at[idx_d0], add=True)
        plsc.subcore_barrier()

        writeback(agg_out)
        zero_acc()
        plsc.subcore_barrier()

        pltpu.sync_copy(dst_hbm.at[pl.ds(base, C)], idx_d0)

        def body2(gg, carry):
            off1 = base + (2 * gg + 1) * C
            off2 = base + (2 * gg + 2) * C
            pltpu.sync_copy(dst_hbm.at[pl.ds(off1, C)], idx_d1)
            pltpu.sync_copy(ones_v, agg_sh.at[idx_d0], add=True)
            pltpu.sync_copy(dst_hbm.at[pl.ds(off2, C)], idx_d0)
            pltpu.sync_copy(ones_v, agg_sh.at[idx_d1], add=True)
            return carry

        lax.fori_loop(0, (gc - 1) // 2, body2, 0)
        pltpu.sync_copy(ones_v, agg_sh.at[idx_d0], add=True)
        plsc.subcore_barrier()

        writeback(deg_out)

    return sc_kern(x, src, dst, z_agg)


def _tc_linear_stats(agg_p, deg_p, W, block):
    _, n, d = agg_p.shape
    nb = n // block

    def body(p_ref, d_ref, w_ref, h_ref, s_ref, q_ref):
        i = pl.program_id(0)
        agg = p_ref[0] + p_ref[1]
        deg = d_ref[0, :, 0:1] + d_ref[1, :, 0:1]
        deg = jnp.maximum(deg, 1.0)
        h = jnp.dot(agg / deg, w_ref[...], preferred_element_type=jnp.float32)
        h_ref[...] = h
        s8 = jnp.sum(h.reshape(block // 8, 8, d), axis=0)
        q8 = jnp.sum((h * h).reshape(block // 8, 8, d), axis=0)

        @pl.when(i == 0)
        def _():
            s_ref[...] = s8
            q_ref[...] = q8

        @pl.when(i != 0)
        def _():
            s_ref[...] += s8
            q_ref[...] += q8

    return pl.pallas_call(
        body,
        grid=(nb,),
        in_specs=[
            pl.BlockSpec((NC, block, d), lambda i: (0, i, 0)),
            pl.BlockSpec((NC, block, d), lambda i: (0, i, 0)),
            pl.BlockSpec((d, d), lambda i: (0, 0)),
        ],
        out_specs=[
            pl.BlockSpec((block, d), lambda i: (i, 0)),
            pl.BlockSpec((8, d), lambda i: (0, 0)),
            pl.BlockSpec((8, d), lambda i: (0, 0)),
        ],
        out_shape=[
            jax.ShapeDtypeStruct((n, d), jnp.float32),
            jax.ShapeDtypeStruct((8, d), jnp.float32),
            jax.ShapeDtypeStruct((8, d), jnp.float32),
        ],
    )(agg_p, deg_p, W)


def _tc_bn_gru(h_pre, h_prev, sums, sumsq, gamma, beta,
               Wz, Uz, bz, Wr, Ur, br, Wh, Uh, bh, block):
    n, d = h_pre.shape
    nb = n // block
    inv_n = 1.0 / n

    def body(h_ref, hp_ref, s_ref, q_ref, g_ref, b_ref,
             wz_ref, uz_ref, bz_ref, wr_ref, ur_ref, br_ref,
             wh_ref, uh_ref, bh_ref, o_ref):
        mu = jnp.sum(s_ref[...], axis=0, keepdims=True) * inv_n
        ex2 = jnp.sum(q_ref[...], axis=0, keepdims=True) * inv_n
        var = ex2 - mu * mu
        inv = lax.rsqrt(var + 1e-5)
        h = (h_ref[...] - mu) * (inv * g_ref[...]) + b_ref[...]
        h = jnp.maximum(h, 0.0)
        hp = hp_ref[...]
        dot = lambda a, b: jnp.dot(a, b, preferred_element_type=jnp.float32)
        z = jax.nn.sigmoid(dot(h, wz_ref[...]) + dot(hp, uz_ref[...])
                           + bz_ref[...])
        r = jax.nn.sigmoid(dot(h, wr_ref[...]) + dot(hp, ur_ref[...])
                           + br_ref[...])
        ht = jnp.tanh(dot(h, wh_ref[...]) + dot(r * hp, uh_ref[...])
                      + bh_ref[...])
        o_ref[...] = (1.0 - z) * hp + z * ht

    blk = pl.BlockSpec((block, d), lambda i: (i, 0))
    small = pl.BlockSpec((8, d), lambda i: (0, 0))
    row = pl.BlockSpec((1, d), lambda i: (0, 0))
    mat = pl.BlockSpec((d, d), lambda i: (0, 0))
    return pl.pallas_call(
        body,
        grid=(nb,),
        in_specs=[blk, blk, small, small, row, row,
                  mat, mat, row, mat, mat, row, mat, mat, row],
        out_specs=blk,
        out_shape=jax.ShapeDtypeStruct((n, d), jnp.float32),
    )(h_pre, h_prev, sums, sumsq, gamma.reshape(1, d), beta.reshape(1, d),
      Wz, Uz, bz.reshape(1, d), Wr, Ur, br.reshape(1, d),
      Wh, Uh, bh.reshape(1, d))


def kernel(x, edge_index, h_prev, W, gamma, beta,
           Wz, Uz, bz, Wr, Ur, br, Wh, Uh, bh):
    n, d = x.shape
    e = edge_index.shape[1]
    assert e % (NW * C) == 0 and n % 8 == 0

    src = edge_index[0].astype(jnp.int32)
    dst = edge_index[1].astype(jnp.int32)
    z_agg = jnp.zeros((n, d), jnp.float32)

    agg_p, deg_p = _sc_aggregate(x, src, dst, z_agg)

    block = 1000
    h_pre, sums, sumsq = _tc_linear_stats(agg_p, deg_p, W, block)
    return _tc_bn_gru(h_pre, h_prev, sums, sumsq, gamma, beta,
                      Wz, Uz, bz, Wr, Ur, br, Wh, Uh, bh, block)


# speedup vs baseline: 1.7635x; 1.0113x over previous
import functools

import jax
import jax.numpy as jnp
from jax import lax
from jax.experimental import pallas as pl
from jax.experimental.pallas import tpu as pltpu
from jax.experimental.pallas import tpu_sc as plsc

NC = 2
NS = 16
NW = NC * NS
C = 80
L = 16


def _sc_aggregate(x, src, dst, z_agg):
    n, d = x.shape
    e = src.shape[0]
    ept = e // NW
    gc = ept // C
    rpt = (n // NS) // 8 * 8
    rlast = n - (NS - 1) * rpt
    assert gc % 2 == 1

    mesh = plsc.VectorSubcoreMesh(core_axis_name="c", subcore_axis_name="s")

    @functools.partial(
        pl.kernel,
        mesh=mesh,
        out_type=[
            jax.ShapeDtypeStruct((NC, n, d), jnp.float32),
            jax.ShapeDtypeStruct((NC, n, d), jnp.float32),
        ],
        scratch_types=[
            pltpu.VMEM((C,), jnp.int32),
            pltpu.VMEM((C,), jnp.int32),
            pltpu.VMEM((C,), jnp.int32),
            pltpu.VMEM((C,), jnp.int32),
            pltpu.VMEM((C, d), jnp.float32),
            pltpu.VMEM((C, d), jnp.float32),
            pltpu.VMEM((C, d), jnp.float32),
            pltpu.VMEM_SHARED((n, d), jnp.float32),
            pltpu.SemaphoreType.DMA,
            pltpu.SemaphoreType.DMA,
        ],
    )
    def sc_kern(x_hbm, src_hbm, dst_hbm, zagg_hbm, agg_out, deg_out,
                idx_s0, idx_s1, idx_d0, idx_d1, rows0, rows1, ones_v,
                agg_sh, sem0, sem1):
        c = lax.axis_index("c")
        s = lax.axis_index("s")
        r0 = s * rpt
        base = (c * NS + s) * ept

        def zero_acc():
            @pl.when(s < NS - 1)
            def _():
                pltpu.sync_copy(zagg_hbm.at[pl.ds(r0, rpt)],
                                agg_sh.at[pl.ds(r0, rpt)])

            @pl.when(s == NS - 1)
            def _():
                pltpu.sync_copy(zagg_hbm.at[pl.ds(r0, rlast)],
                                agg_sh.at[pl.ds(r0, rlast)])

        def writeback(out_ref):
            @pl.when(s < NS - 1)
            def _():
                pltpu.sync_copy(agg_sh.at[pl.ds(r0, rpt)],
                                out_ref.at[c, pl.ds(r0, rpt)])

            @pl.when(s == NS - 1)
            def _():
                pltpu.sync_copy(agg_sh.at[pl.ds(r0, rlast)],
                                out_ref.at[c, pl.ds(r0, rlast)])

        zero_acc()

        def fill_row(i, carry):
            def fill_lane(j, carry2):
                ones_v[i, pl.ds(j * L, L)] = jnp.full((L,), 1.0, jnp.float32)
                return carry2
            return lax.fori_loop(0, d // L, fill_lane, carry)

        lax.fori_loop(0, C, fill_row, 0)
        plsc.subcore_barrier()

        pltpu.sync_copy(src_hbm.at[pl.ds(base, C)], idx_s0)
        pltpu.sync_copy(dst_hbm.at[pl.ds(base, C)], idx_d0)
        pltpu.async_copy(x_hbm.at[idx_s0], rows0, sem0)

        def body1(gg, carry):
            off1 = base + (2 * gg + 1) * C
            off2 = base + (2 * gg + 2) * C
            pltpu.sync_copy(src_hbm.at[pl.ds(off1, C)], idx_s1)
            pltpu.sync_copy(dst_hbm.at[pl.ds(off1, C)], idx_d1)
            pltpu.async_copy(x_hbm.at[idx_s1], rows1, sem1)
            pltpu.make_async_copy(x_hbm.at[idx_s0], rows0, sem0).wait()
            pltpu.sync_copy(rows0, agg_sh.at[idx_d0], add=True)
            pltpu.sync_copy(src_hbm.at[pl.ds(off2, C)], idx_s0)
            pltpu.sync_copy(dst_hbm.at[pl.ds(off2, C)], idx_d0)
            pltpu.async_copy(x_hbm.at[idx_s0], rows0, sem0)
            pltpu.make_async_copy(x_hbm.at[idx_s1], rows1, sem1).wait()
            pltpu.sync_copy(rows1, agg_sh.at[idx_d1], add=True)
            return carry

        lax.fori_loop(0, (gc - 1) // 2, body1, 0)
        pltpu.make_async_copy(x_hbm.at[idx_s0], rows0, sem0).wait()
        pltpu.sync_copy(rows0, agg_sh.at[idx_d0], add=True)
        plsc.subcore_barrier()

        writeback(agg_out)
        zero_acc()
        plsc.subcore_barrier()

        pltpu.sync_copy(dst_hbm.at[pl.ds(base, C)], idx_d0)

        def body2(gg, carry):
            off1 = base + (2 * gg + 1) * C
            off2 = base + (2 * gg + 2) * C
            pltpu.sync_copy(dst_hbm.at[pl.ds(off1, C)], idx_d1)
            pltpu.async_copy(ones_v, agg_sh.at[idx_d0], sem0, add=True)
            pltpu.async_copy(ones_v, agg_sh.at[idx_d1], sem1, add=True)
            pltpu.make_async_copy(ones_v, agg_sh.at[idx_d0], sem0).wait()
            pltpu.sync_copy(dst_hbm.at[pl.ds(off2, C)], idx_d0)
            pltpu.make_async_copy(ones_v, agg_sh.at[idx_d1], sem1).wait()
            return carry

        lax.fori_loop(0, (gc - 1) // 2, body2, 0)
        pltpu.sync_copy(ones_v, agg_sh.at[idx_d0], add=True)
        plsc.subcore_barrier()

        writeback(deg_out)

    return sc_kern(x, src, dst, z_agg)


def _tc_linear_stats(agg_p, deg_p, W, block):
    _, n, d = agg_p.shape
    nb = n // block

    def body(p_ref, d_ref, w_ref, h_ref, s_ref, q_ref):
        i = pl.program_id(0)
        agg = p_ref[0] + p_ref[1]
        deg = d_ref[0, :, 0:1] + d_ref[1, :, 0:1]
        deg = jnp.maximum(deg, 1.0)
        h = jnp.dot(agg / deg, w_ref[...], preferred_element_type=jnp.float32)
        h_ref[...] = h
        s8 = jnp.sum(h.reshape(block // 8, 8, d), axis=0)
        q8 = jnp.sum((h * h).reshape(block // 8, 8, d), axis=0)

        @pl.when(i == 0)
        def _():
            s_ref[...] = s8
            q_ref[...] = q8

        @pl.when(i != 0)
        def _():
            s_ref[...] += s8
            q_ref[...] += q8

    return pl.pallas_call(
        body,
        grid=(nb,),
        in_specs=[
            pl.BlockSpec((NC, block, d), lambda i: (0, i, 0)),
            pl.BlockSpec((NC, block, d), lambda i: (0, i, 0)),
            pl.BlockSpec((d, d), lambda i: (0, 0)),
        ],
        out_specs=[
            pl.BlockSpec((block, d), lambda i: (i, 0)),
            pl.BlockSpec((8, d), lambda i: (0, 0)),
            pl.BlockSpec((8, d), lambda i: (0, 0)),
        ],
        out_shape=[
            jax.ShapeDtypeStruct((n, d), jnp.float32),
            jax.ShapeDtypeStruct((8, d), jnp.float32),
            jax.ShapeDtypeStruct((8, d), jnp.float32),
        ],
    )(agg_p, deg_p, W)


def _tc_bn_gru(h_pre, h_prev, sums, sumsq, gamma, beta,
               Wz, Uz, bz, Wr, Ur, br, Wh, Uh, bh, block):
    n, d = h_pre.shape
    nb = n // block
    inv_n = 1.0 / n

    def body(h_ref, hp_ref, s_ref, q_ref, g_ref, b_ref,
             wz_ref, uz_ref, bz_ref, wr_ref, ur_ref, br_ref,
             wh_ref, uh_ref, bh_ref, o_ref):
        mu = jnp.sum(s_ref[...], axis=0, keepdims=True) * inv_n
        ex2 = jnp.sum(q_ref[...], axis=0, keepdims=True) * inv_n
        var = ex2 - mu * mu
        inv = lax.rsqrt(var + 1e-5)
        h = (h_ref[...] - mu) * (inv * g_ref[...]) + b_ref[...]
        h = jnp.maximum(h, 0.0)
        hp = hp_ref[...]
        dot = lambda a, b: jnp.dot(a, b, preferred_element_type=jnp.float32)
        z = jax.nn.sigmoid(dot(h, wz_ref[...]) + dot(hp, uz_ref[...])
                           + bz_ref[...])
        r = jax.nn.sigmoid(dot(h, wr_ref[...]) + dot(hp, ur_ref[...])
                           + br_ref[...])
        ht = jnp.tanh(dot(h, wh_ref[...]) + dot(r * hp, uh_ref[...])
                      + bh_ref[...])
        o_ref[...] = (1.0 - z) * hp + z * ht

    blk = pl.BlockSpec((block, d), lambda i: (i, 0))
    small = pl.BlockSpec((8, d), lambda i: (0, 0))
    row = pl.BlockSpec((1, d), lambda i: (0, 0))
    mat = pl.BlockSpec((d, d), lambda i: (0, 0))
    return pl.pallas_call(
        body,
        grid=(nb,),
        in_specs=[blk, blk, small, small, row, row,
                  mat, mat, row, mat, mat, row, mat, mat, row],
        out_specs=blk,
        out_shape=jax.ShapeDtypeStruct((n, d), jnp.float32),
    )(h_pre, h_prev, sums, sumsq, gamma.reshape(1, d), beta.reshape(1, d),
      Wz, Uz, bz.reshape(1, d), Wr, Ur, br.reshape(1, d),
      Wh, Uh, bh.reshape(1, d))


def kernel(x, edge_index, h_prev, W, gamma, beta,
           Wz, Uz, bz, Wr, Ur, br, Wh, Uh, bh):
    n, d = x.shape
    e = edge_index.shape[1]
    assert e % (NW * C) == 0 and n % 8 == 0

    src = edge_index[0].astype(jnp.int32)
    dst = edge_index[1].astype(jnp.int32)
    z_agg = jnp.zeros((n, d), jnp.float32)

    agg_p, deg_p = _sc_aggregate(x, src, dst, z_agg)

    block = 1000
    h_pre, sums, sumsq = _tc_linear_stats(agg_p, deg_p, W, block)
    return _tc_bn_gru(h_pre, h_prev, sums, sumsq, gamma, beta,
                      Wz, Uz, bz, Wr, Ur, br, Wh, Uh, bh, block)


# speedup vs baseline: 2.3019x; 1.3053x over previous
import functools

import jax
import jax.numpy as jnp
from jax import lax
from jax.experimental import pallas as pl
from jax.experimental.pallas import tpu as pltpu
from jax.experimental.pallas import tpu_sc as plsc

NC = 2
NS = 16
NW = NC * NS
C = 80
L = 16


def _sc_aggregate(x, sd_flat, z_agg, e):
    n, d = x.shape
    ept = e // NW
    gc = ept // C
    C2 = 2 * C
    rpt = (n // NS) // 8 * 8
    rlast = n - (NS - 1) * rpt
    assert gc % 2 == 1

    mesh = plsc.VectorSubcoreMesh(core_axis_name="c", subcore_axis_name="s")

    @functools.partial(
        pl.kernel,
        mesh=mesh,
        out_type=[
            jax.ShapeDtypeStruct((NC, n, d), jnp.float32),
            jax.ShapeDtypeStruct((NC, n, d), jnp.float32),
        ],
        scratch_types=[
            pltpu.VMEM((C2,), jnp.int32),
            pltpu.VMEM((C2,), jnp.int32),
            pltpu.VMEM((C,), jnp.int32),
            pltpu.VMEM((C,), jnp.int32),
            pltpu.VMEM((C, d), jnp.float32),
            pltpu.VMEM((C, d), jnp.float32),
            pltpu.VMEM((C, d), jnp.float32),
            pltpu.VMEM_SHARED((n, d), jnp.float32),
            pltpu.SemaphoreType.DMA,
            pltpu.SemaphoreType.DMA,
            pltpu.SemaphoreType.DMA,
            pltpu.SemaphoreType.DMA,
        ],
    )
    def sc_kern(x_hbm, sd_hbm, zagg_hbm, agg_out, deg_out,
                ibuf0, ibuf1, idx_d0, idx_d1, rows0, rows1, ones_v,
                agg_sh, sem0, sem1, semI0, semI1):
        c = lax.axis_index("c")
        s = lax.axis_index("s")
        r0 = s * rpt
        qbase = (c * NS + s) * gc

        def iload(q, ibuf, semI):
            return pltpu.async_copy(sd_hbm.at[pl.ds(q * C2, C2)], ibuf, semI)

        def iload_wait(ibuf, semI):
            pltpu.make_async_copy(sd_hbm.at[pl.ds(0, C2)], ibuf, semI).wait()

        def bounce(ibuf, idx_d):
            for k in range(C // L):
                idx_d[pl.ds(k * L, L)] = ibuf[pl.ds(C + k * L, L)]

        def zero_acc():
            @pl.when(s < NS - 1)
            def _():
                pltpu.sync_copy(zagg_hbm.at[pl.ds(r0, rpt)],
                                agg_sh.at[pl.ds(r0, rpt)])

            @pl.when(s == NS - 1)
            def _():
                pltpu.sync_copy(zagg_hbm.at[pl.ds(r0, rlast)],
                                agg_sh.at[pl.ds(r0, rlast)])

        def writeback(out_ref):
            @pl.when(s < NS - 1)
            def _():
                pltpu.sync_copy(agg_sh.at[pl.ds(r0, rpt)],
                                out_ref.at[c, pl.ds(r0, rpt)])

            @pl.when(s == NS - 1)
            def _():
                pltpu.sync_copy(agg_sh.at[pl.ds(r0, rlast)],
                                out_ref.at[c, pl.ds(r0, rlast)])

        zero_acc()

        def fill_row(i, carry):
            def fill_lane(j, carry2):
                ones_v[i, pl.ds(j * L, L)] = jnp.full((L,), 1.0, jnp.float32)
                return carry2
            return lax.fori_loop(0, d // L, fill_lane, carry)

        lax.fori_loop(0, C, fill_row, 0)
        plsc.subcore_barrier()

        def gather(ibuf, rows, sem):
            return pltpu.async_copy(x_hbm.at[ibuf.at[pl.ds(0, C)]],
                                    rows, sem)

        def gather_wait(rows, sem):
            pltpu.make_async_copy(x_hbm.at[ibuf0.at[pl.ds(0, C)]],
                                  rows, sem).wait()

        iload(qbase, ibuf0, semI0)
        iload_wait(ibuf0, semI0)
        bounce(ibuf0, idx_d0)
        gather(ibuf0, rows0, sem0)
        iload(qbase + 1, ibuf1, semI1)

        last = qbase + gc - 1

        def body1(gg, carry):
            g0 = qbase + 2 * gg
            iload_wait(ibuf1, semI1)
            bounce(ibuf1, idx_d1)
            gather(ibuf1, rows1, sem1)
            gather_wait(rows0, sem0)
            iload(g0 + 2, ibuf0, semI0)
            pltpu.sync_copy(rows0, agg_sh.at[idx_d0], add=True)
            iload_wait(ibuf0, semI0)
            bounce(ibuf0, idx_d0)
            gather(ibuf0, rows0, sem0)
            gather_wait(rows1, sem1)
            iload(jnp.minimum(g0 + 3, last), ibuf1, semI1)
            pltpu.sync_copy(rows1, agg_sh.at[idx_d1], add=True)
            return carry

        lax.fori_loop(0, (gc - 1) // 2, body1, 0)
        gather_wait(rows0, sem0)
        pltpu.sync_copy(rows0, agg_sh.at[idx_d0], add=True)
        iload_wait(ibuf1, semI1)
        plsc.subcore_barrier()

        writeback(agg_out)
        zero_acc()
        plsc.subcore_barrier()

        iload(qbase, ibuf0, semI0)
        iload_wait(ibuf0, semI0)
        bounce(ibuf0, idx_d0)
        iload(qbase + 2, ibuf0, semI0)
        iload(qbase + 1, ibuf1, semI1)

        def body2(gg, carry):
            g0 = qbase + 2 * gg
            iload_wait(ibuf1, semI1)
            bounce(ibuf1, idx_d1)
            pltpu.async_copy(ones_v, agg_sh.at[idx_d0], sem0, add=True)
            pltpu.async_copy(ones_v, agg_sh.at[idx_d1], sem1, add=True)
            iload(jnp.minimum(g0 + 3, last), ibuf1, semI1)
            pltpu.make_async_copy(ones_v, agg_sh.at[idx_d0], sem0).wait()
            iload_wait(ibuf0, semI0)
            bounce(ibuf0, idx_d0)
            iload(jnp.minimum(g0 + 4, last), ibuf0, semI0)
            pltpu.make_async_copy(ones_v, agg_sh.at[idx_d1], sem1).wait()
            return carry

        lax.fori_loop(0, (gc - 1) // 2, body2, 0)
        pltpu.sync_copy(ones_v, agg_sh.at[idx_d0], add=True)
        iload_wait(ibuf0, semI0)
        iload_wait(ibuf1, semI1)
        plsc.subcore_barrier()

        writeback(deg_out)

    return sc_kern(x, sd_flat, z_agg)


def _tc_linear_stats(agg_p, deg_p, W, block):
    _, n, d = agg_p.shape
    nb = n // block

    def body(p_ref, d_ref, w_ref, h_ref, s_ref, q_ref):
        i = pl.program_id(0)
        agg = p_ref[0] + p_ref[1]
        deg = d_ref[0, :, 0:1] + d_ref[1, :, 0:1]
        deg = jnp.maximum(deg, 1.0)
        h = jnp.dot(agg / deg, w_ref[...], preferred_element_type=jnp.float32)
        h_ref[...] = h
        s8 = jnp.sum(h.reshape(block // 8, 8, d), axis=0)
        q8 = jnp.sum((h * h).reshape(block // 8, 8, d), axis=0)

        @pl.when(i == 0)
        def _():
            s_ref[...] = s8
            q_ref[...] = q8

        @pl.when(i != 0)
        def _():
            s_ref[...] += s8
            q_ref[...] += q8

    return pl.pallas_call(
        body,
        grid=(nb,),
        in_specs=[
            pl.BlockSpec((NC, block, d), lambda i: (0, i, 0)),
            pl.BlockSpec((NC, block, d), lambda i: (0, i, 0)),
            pl.BlockSpec((d, d), lambda i: (0, 0)),
        ],
        out_specs=[
            pl.BlockSpec((block, d), lambda i: (i, 0)),
            pl.BlockSpec((8, d), lambda i: (0, 0)),
            pl.BlockSpec((8, d), lambda i: (0, 0)),
        ],
        out_shape=[
            jax.ShapeDtypeStruct((n, d), jnp.float32),
            jax.ShapeDtypeStruct((8, d), jnp.float32),
            jax.ShapeDtypeStruct((8, d), jnp.float32),
        ],
    )(agg_p, deg_p, W)


def _tc_bn_gru(h_pre, h_prev, sums, sumsq, gamma, beta,
               Wz, Uz, bz, Wr, Ur, br, Wh, Uh, bh, block):
    n, d = h_pre.shape
    nb = n // block
    inv_n = 1.0 / n

    def body(h_ref, hp_ref, s_ref, q_ref, g_ref, b_ref,
             wz_ref, uz_ref, bz_ref, wr_ref, ur_ref, br_ref,
             wh_ref, uh_ref, bh_ref, o_ref):
        mu = jnp.sum(s_ref[...], axis=0, keepdims=True) * inv_n
        ex2 = jnp.sum(q_ref[...], axis=0, keepdims=True) * inv_n
        var = ex2 - mu * mu
        inv = lax.rsqrt(var + 1e-5)
        h = (h_ref[...] - mu) * (inv * g_ref[...]) + b_ref[...]
        h = jnp.maximum(h, 0.0)
        hp = hp_ref[...]
        dot = lambda a, b: jnp.dot(a, b, preferred_element_type=jnp.float32)
        z = jax.nn.sigmoid(dot(h, wz_ref[...]) + dot(hp, uz_ref[...])
                           + bz_ref[...])
        r = jax.nn.sigmoid(dot(h, wr_ref[...]) + dot(hp, ur_ref[...])
                           + br_ref[...])
        ht = jnp.tanh(dot(h, wh_ref[...]) + dot(r * hp, uh_ref[...])
                      + bh_ref[...])
        o_ref[...] = (1.0 - z) * hp + z * ht

    blk = pl.BlockSpec((block, d), lambda i: (i, 0))
    small = pl.BlockSpec((8, d), lambda i: (0, 0))
    row = pl.BlockSpec((1, d), lambda i: (0, 0))
    mat = pl.BlockSpec((d, d), lambda i: (0, 0))
    return pl.pallas_call(
        body,
        grid=(nb,),
        in_specs=[blk, blk, small, small, row, row,
                  mat, mat, row, mat, mat, row, mat, mat, row],
        out_specs=blk,
        out_shape=jax.ShapeDtypeStruct((n, d), jnp.float32),
    )(h_pre, h_prev, sums, sumsq, gamma.reshape(1, d), beta.reshape(1, d),
      Wz, Uz, bz.reshape(1, d), Wr, Ur, br.reshape(1, d),
      Wh, Uh, bh.reshape(1, d))


def kernel(x, edge_index, h_prev, W, gamma, beta,
           Wz, Uz, bz, Wr, Ur, br, Wh, Uh, bh):
    n, d = x.shape
    e = edge_index.shape[1]
    assert e % (NW * C) == 0 and n % 8 == 0

    src = edge_index[0].astype(jnp.int32)
    dst = edge_index[1].astype(jnp.int32)
    q = e // C
    sd_flat = jnp.concatenate(
        [src.reshape(q, 1, C), dst.reshape(q, 1, C)], axis=1).reshape(-1)
    z_agg = jnp.zeros((n, d), jnp.float32)

    agg_p, deg_p = _sc_aggregate(x, sd_flat, z_agg, e)

    block = 1000
    h_pre, sums, sumsq = _tc_linear_stats(agg_p, deg_p, W, block)
    return _tc_bn_gru(h_pre, h_prev, sums, sumsq, gamma, beta,
                      Wz, Uz, bz, Wr, Ur, br, Wh, Uh, bh, block)
